# Initial kernel scaffold; baseline (speedup 1.0000x reference)
import jax
import jax.numpy as jnp
from jax.experimental import pallas as pl


def kernel(x, edge_index, W1, b1, W2, b2, W3, b3):
    raise NotImplementedError("write your pallas kernel here")


# speedup vs baseline: 14.0036x; 14.0036x over previous
import functools

import jax
import jax.numpy as jnp
from jax import lax
from jax.experimental import pallas as pl
from jax.experimental.pallas import tpu as pltpu
from jax.experimental.pallas import tpu_sc as plsc

_N = 10000
_E = 320000
_D = 128
_CH = 128
_NCHUNK = _E // _CH
_NP = 10112
_NROWCH = _NP // _CH
_NC = 2
_NS = 16
_NW = _NC * _NS


def _sc_mesh():
    return plsc.VectorSubcoreMesh(
        core_axis_name="c", subcore_axis_name="s", num_cores=_NC, num_subcores=_NS
    )


@functools.partial(
    pl.kernel,
    out_type=jax.ShapeDtypeStruct((_NC, _NP), jnp.float32),
    mesh=_sc_mesh(),
    scratch_types=[
        pltpu.VMEM((_CH,), jnp.int32),
        pltpu.VMEM((_CH,), jnp.float32),
        pltpu.VMEM((_CH,), jnp.float32),
        pltpu.VMEM_SHARED((_NP,), jnp.float32),
    ],
)
def _sc_degree(col_hbm, out_hbm, col_v, ones_v, zero_v, accum):
    cid = lax.axis_index("c")
    sid = lax.axis_index("s")
    wid = cid * _NS + sid
    for j in range(_CH // 16):
        ones_v[pl.ds(j * 16, 16)] = jnp.ones((16,), jnp.float32)
        zero_v[pl.ds(j * 16, 16)] = jnp.zeros((16,), jnp.float32)
    for k in range(5):
        c = sid + k * _NS

        @pl.when(c < _NROWCH)
        def _():
            pltpu.sync_copy(zero_v, accum.at[pl.ds(c * _CH, _CH)])

    plsc.subcore_barrier()

    def body(k, carry):
        c = wid + k * _NW

        @pl.when(c < _NCHUNK)
        def _():
            pltpu.sync_copy(col_hbm.at[pl.ds(c * _CH, _CH)], col_v)
            pltpu.sync_copy(ones_v, accum.at[col_v], add=True)

        return carry

    lax.fori_loop(0, (_NCHUNK + _NW - 1) // _NW, body, 0)
    plsc.subcore_barrier()
    for k in range(5):
        c = sid + k * _NS

        @pl.when(c < _NROWCH)
        def _():
            pltpu.sync_copy(
                accum.at[pl.ds(c * _CH, _CH)], out_hbm.at[cid, pl.ds(c * _CH, _CH)]
            )


@functools.partial(
    pl.kernel,
    out_type=jax.ShapeDtypeStruct((_NC, _NP, _D), jnp.float32),
    mesh=_sc_mesh(),
    scratch_types=[
        pltpu.VMEM((_CH,), jnp.int32),
        pltpu.VMEM((_CH,), jnp.int32),
        pltpu.VMEM((_CH, _D), jnp.float32),
        pltpu.VMEM_SHARED((_NP, _D), jnp.float32),
        pltpu.SemaphoreType.DMA,
    ],
)
def _sc_agg(hs_hbm, row_hbm, col_hbm, out_hbm, row_v, col_v, rows_v, accum, sem):
    cid = lax.axis_index("c")
    sid = lax.axis_index("s")
    wid = cid * _NS + sid

    def zbody(i, carry):
        for j in range(_D // 16):
            rows_v[i, pl.ds(j * 16, 16)] = jnp.zeros((16,), jnp.float32)
        return carry

    lax.fori_loop(0, _CH, zbody, 0)
    for k in range(5):
        c = sid + k * _NS

        @pl.when(c < _NROWCH)
        def _():
            pltpu.sync_copy(rows_v, accum.at[pl.ds(c * _CH, _CH)])

    plsc.subcore_barrier()

    def body(k, carry):
        c = wid + k * _NW

        @pl.when(c < _NCHUNK)
        def _():
            pltpu.sync_copy(row_hbm.at[pl.ds(c * _CH, _CH)], row_v)
            pltpu.async_copy(hs_hbm.at[row_v], rows_v, sem).wait()
            pltpu.sync_copy(col_hbm.at[pl.ds(c * _CH, _CH)], col_v)
            pltpu.sync_copy(rows_v, accum.at[col_v], add=True)

        return carry

    lax.fori_loop(0, (_NCHUNK + _NW - 1) // _NW, body, 0)
    plsc.subcore_barrier()
    for k in range(5):
        c = sid + k * _NS

        @pl.when(c < _NROWCH)
        def _():
            pltpu.sync_copy(
                accum.at[pl.ds(c * _CH, _CH)], out_hbm.at[cid, pl.ds(c * _CH, _CH)]
            )


_BN = 1000


def _tc_first_body(dp_ref, x_ref, wt_ref, dis_ref, hs_ref):
    d = jnp.sum(dp_ref[...], axis=1, keepdims=True) + 1.0
    dis = lax.rsqrt(d)
    h = jnp.dot(
        x_ref[...], wt_ref[...],
        preferred_element_type=jnp.float32,
        precision=lax.Precision.HIGHEST,
    )
    dis_ref[...] = dis
    hs_ref[...] = h * dis


def _tc_first(degp_t, x, w1t):
    return pl.pallas_call(
        _tc_first_body,
        grid=(_N // _BN,),
        in_specs=[
            pl.BlockSpec((_BN, 2), lambda i: (i, 0)),
            pl.BlockSpec((_BN, _D), lambda i: (i, 0)),
            pl.BlockSpec((_D, _D), lambda i: (0, 0)),
        ],
        out_specs=[
            pl.BlockSpec((_BN, 1), lambda i: (i, 0)),
            pl.BlockSpec((_BN, _D), lambda i: (i, 0)),
        ],
        out_shape=[
            jax.ShapeDtypeStruct((_N, 1), jnp.float32),
            jax.ShapeDtypeStruct((_N, _D), jnp.float32),
        ],
    )(degp_t, x, w1t)


def _tc_mid_body(p_ref, hs_ref, dis_ref, b_ref, wt_ref, out_ref):
    pblk = p_ref[...]
    agg = pblk[0] + pblk[1] + hs_ref[...]
    dis = dis_ref[...]
    o = agg * dis + b_ref[...]
    a = jnp.maximum(o, 0.0)
    h2 = jnp.dot(
        a, wt_ref[...],
        preferred_element_type=jnp.float32,
        precision=lax.Precision.HIGHEST,
    )
    out_ref[...] = h2 * dis


def _tc_mid(p, hs, dis, b, wt):
    return pl.pallas_call(
        _tc_mid_body,
        grid=(_N // _BN,),
        in_specs=[
            pl.BlockSpec((_NC, _BN, _D), lambda i: (0, i, 0)),
            pl.BlockSpec((_BN, _D), lambda i: (i, 0)),
            pl.BlockSpec((_BN, 1), lambda i: (i, 0)),
            pl.BlockSpec((1, _D), lambda i: (0, 0)),
            pl.BlockSpec((_D, _D), lambda i: (0, 0)),
        ],
        out_specs=pl.BlockSpec((_BN, _D), lambda i: (i, 0)),
        out_shape=jax.ShapeDtypeStruct((_N, _D), jnp.float32),
    )(p, hs, dis, b, wt)


def _tc_last_body(p_ref, hs_ref, dis_ref, b_ref, out_ref):
    pblk = p_ref[...]
    agg = pblk[0] + pblk[1] + hs_ref[...]
    out_ref[...] = agg * dis_ref[...] + b_ref[...]


def _tc_last(p, hs, dis, b):
    return pl.pallas_call(
        _tc_last_body,
        grid=(_N // _BN,),
        in_specs=[
            pl.BlockSpec((_NC, _BN, _D), lambda i: (0, i, 0)),
            pl.BlockSpec((_BN, _D), lambda i: (i, 0)),
            pl.BlockSpec((_BN, 1), lambda i: (i, 0)),
            pl.BlockSpec((1, _D), lambda i: (0, 0)),
        ],
        out_specs=pl.BlockSpec((_BN, _D), lambda i: (i, 0)),
        out_shape=jax.ShapeDtypeStruct((_N, _D), jnp.float32),
    )(p, hs, dis, b)


def kernel(x, edge_index, W1, b1, W2, b2, W3, b3):
    row = edge_index[0]
    col = edge_index[1]
    degp = _sc_degree(col)
    dis, hs1 = _tc_first(degp.T, x, W1.T)
    p = _sc_agg(hs1, row, col)
    hs2 = _tc_mid(p, hs1, dis, b1.reshape(1, _D), W2.T)
    p = _sc_agg(hs2, row, col)
    hs3 = _tc_mid(p, hs2, dis, b2.reshape(1, _D), W3.T)
    p = _sc_agg(hs3, row, col)
    return _tc_last(p, hs3, dis, b3.reshape(1, _D))
